# baseline (speedup 1.0000x reference)
import functools

import jax
import jax.numpy as jnp
from jax import lax
from jax.experimental import pallas as pl
from jax.experimental.pallas import tpu as pltpu
from jax.experimental.pallas import tpu_sc as plsc

VOCAB = 49408
B = 256
L = 77
D = 1024
NVEC = 8

NC = 2
NS = 16
NW = NC * NS
N = B * L
PER_W = N // NW
BPW = B // NW
CHUNK = 48
NFULL = PER_W // CHUNK
TAIL = PER_W - NFULL * CHUNK


def _sc_kernel(ids_hbm, table_hbm, ti2_hbm, dest_hbm, out_hbm,
               ids_v, buf0, buf1, ti_v, didx_v,
               gsem0, gsem1, wsem0, wsem1, dsem):
    wid = lax.axis_index("s") * NC + lax.axis_index("c")
    base = wid * PER_W

    pltpu.sync_copy(ids_hbm.at[pl.ds(base, PER_W)], ids_v)
    pltpu.sync_copy(ti2_hbm, ti_v)

    bufs = (buf0, buf1)
    gsems = (gsem0, gsem1)
    wsems = (wsem0, wsem1)
    nchunks = NFULL + 1
    sizes = [CHUNK] * NFULL + [TAIL]

    def gather(c):
        return pltpu.async_copy(
            table_hbm.at[ids_v.at[pl.ds(c * CHUNK, sizes[c])]],
            bufs[c % 2].at[pl.ds(0, sizes[c])], gsems[c % 2])

    def writeback(c):
        return pltpu.async_copy(
            bufs[c % 2].at[pl.ds(0, sizes[c])],
            out_hbm.at[pl.ds(base + c * CHUNK, sizes[c])], wsems[c % 2])

    g = [None] * nchunks
    w = [None] * nchunks
    for c in range(nchunks):
        if c >= 2:
            w[c - 2].wait()
        g[c] = gather(c)
        if c >= 1:
            g[c - 1].wait()
            w[c - 1] = writeback(c - 1)
    g[nchunks - 1].wait()
    w[nchunks - 1] = writeback(nchunks - 1)
    w[nchunks - 2].wait()
    w[nchunks - 1].wait()

    for g in range(BPW // 2):
        pltpu.sync_copy(dest_hbm.at[wid, g], didx_v)
        pltpu.async_copy(ti_v, out_hbm.at[didx_v], dsem).wait()


@jax.jit
def kernel(input_ids, table, ti_emb, offsets):
    ids_flat = input_ids.reshape(N)
    ti2 = jnp.concatenate([ti_emb, ti_emb], axis=0)
    lane = jnp.arange(16, dtype=jnp.int32)
    g = jnp.arange(BPW // 2, dtype=jnp.int32)
    lb = g[None, :, None] * 2 + (lane[None, None, :] >> 3)
    prompt = jnp.arange(NW, dtype=jnp.int32)[:, None, None] * BPW + lb
    off = offsets[prompt]
    dest = prompt * L + off + 1 + (lane[None, None, :] & 7)

    mesh = plsc.VectorSubcoreMesh(core_axis_name="c", subcore_axis_name="s")
    out = pl.kernel(
        _sc_kernel,
        out_type=jax.ShapeDtypeStruct((N, D), jnp.float32),
        mesh=mesh,
        scratch_types=[
            pltpu.VMEM((PER_W,), jnp.int32),
            pltpu.VMEM((CHUNK, D), jnp.float32),
            pltpu.VMEM((CHUNK, D), jnp.float32),
            pltpu.VMEM((16, D), jnp.float32),
            pltpu.VMEM((16,), jnp.int32),
            pltpu.SemaphoreType.DMA,
            pltpu.SemaphoreType.DMA,
            pltpu.SemaphoreType.DMA,
            pltpu.SemaphoreType.DMA,
            pltpu.SemaphoreType.DMA,
        ],
    )(ids_flat, table, ti2, dest)
    return out.reshape(B, L, D)


# speedup vs baseline: 2.4100x; 2.4100x over previous
import jax
import jax.numpy as jnp
from jax import lax
from jax.experimental import pallas as pl
from jax.experimental.pallas import tpu as pltpu
from jax.experimental.pallas import tpu_sc as plsc

VOCAB = 49408
B = 256
L = 77
D = 1024
NVEC = 8

NC = 2
NS = 16
NW = NC * NS
N = B * L
PER_W = N // NW
BPW = B // NW
CHUNK = 48
NFULL = PER_W // CHUNK
TAIL = PER_W - NFULL * CHUNK


def _gather_kernel(ids_hbm, table_hbm, out_hbm,
                   ids_v, buf0, buf1, gsem0, gsem1, wsem0, wsem1):
    wid = lax.axis_index("s") * NC + lax.axis_index("c")
    base = wid * PER_W

    pltpu.sync_copy(ids_hbm.at[pl.ds(base, PER_W)], ids_v)

    bufs = (buf0, buf1)
    gsems = (gsem0, gsem1)
    wsems = (wsem0, wsem1)
    nchunks = NFULL + 1
    sizes = [CHUNK] * NFULL + [TAIL]

    def gather(c):
        return pltpu.async_copy(
            table_hbm.at[ids_v.at[pl.ds(c * CHUNK, sizes[c])]],
            bufs[c % 2].at[pl.ds(0, sizes[c])], gsems[c % 2])

    def writeback(c):
        return pltpu.async_copy(
            bufs[c % 2].at[pl.ds(0, sizes[c])],
            out_hbm.at[pl.ds(base + c * CHUNK, sizes[c])], wsems[c % 2])

    g = [None] * nchunks
    w = [None] * nchunks
    for c in range(nchunks):
        if c >= 2:
            w[c - 2].wait()
        g[c] = gather(c)
        if c >= 1:
            g[c - 1].wait()
            w[c - 1] = writeback(c - 1)
    g[nchunks - 1].wait()
    w[nchunks - 1] = writeback(nchunks - 1)
    w[nchunks - 2].wait()
    w[nchunks - 1].wait()


def _ti_kernel(ti2_hbm, dest_hbm, out_ref, ti_v, didx_v, dsem):
    wid = lax.axis_index("s") * NC + lax.axis_index("c")
    pltpu.sync_copy(ti2_hbm, ti_v)
    for g in range(BPW // 2):
        pltpu.sync_copy(dest_hbm.at[wid, g], didx_v)
        pltpu.async_copy(ti_v, out_ref.at[didx_v], dsem).wait()


@jax.jit
def kernel(input_ids, table, ti_emb, offsets):
    ids_lm = input_ids.T.reshape(N)
    ti2 = jnp.concatenate([ti_emb, ti_emb], axis=0)
    lane = jnp.arange(16, dtype=jnp.int32)
    grp = jnp.arange(BPW // 2, dtype=jnp.int32)
    b = (jnp.arange(NW, dtype=jnp.int32)[:, None, None] * BPW
         + grp[None, :, None] * 2 + (lane[None, None, :] >> 3))
    dest = (offsets[b] + 1 + (lane[None, None, :] & 7)) * B + b

    mesh = plsc.VectorSubcoreMesh(core_axis_name="c", subcore_axis_name="s")
    out2 = pl.kernel(
        _gather_kernel,
        out_type=jax.ShapeDtypeStruct((N, D), jnp.float32),
        mesh=mesh,
        scratch_types=[
            pltpu.VMEM((PER_W,), jnp.int32),
            pltpu.VMEM((CHUNK, D), jnp.float32),
            pltpu.VMEM((CHUNK, D), jnp.float32),
            pltpu.SemaphoreType.DMA,
            pltpu.SemaphoreType.DMA,
            pltpu.SemaphoreType.DMA,
            pltpu.SemaphoreType.DMA,
        ],
    )(ids_lm, table)

    out_ref = jax.new_ref(out2)
    pl.kernel(
        _ti_kernel,
        out_type=(),
        mesh=mesh,
        scratch_types=[
            pltpu.VMEM((16, D), jnp.float32),
            pltpu.VMEM((16,), jnp.int32),
            pltpu.SemaphoreType.DMA,
        ],
    )(ti2, dest, out_ref)
    out = jax.freeze(out_ref)
    return out.reshape(L, B, D).transpose(1, 0, 2)
